# baseline (speedup 1.0000x reference)
import functools

import jax
import jax.numpy as jnp
from jax import lax
from jax.experimental import pallas as pl
from jax.experimental.pallas import tpu as pltpu
from jax.experimental.pallas import tpu_sc as plsc

NUM_CORES = 2
NUM_SUBCORES = 16
NW = NUM_CORES * NUM_SUBCORES
LANES = 16

EMB_DIM = 128
T_DIM = 64
CHUNK = 64
NBUF = 2
NOPND = 7

TS_PITCH = 24


def _tec_body(heads_hbm, rels_hbm, tails_hbm, dates_hbm,
              ent_h_hbm, ent_t_hbm, rel_fi_hbm, tim_hbm,
              out_hbm,
              heads_v, rels_v, tails_v, dates_v,
              h1_0, t1_0, h2_0, t2_0, r12_0, tt_0,
              h1_1, t1_1, h2_1, t2_1, r12_1, tt_1,
              ts_v, out_v,
              sem0, sem1):
    b_per_w = heads_v.shape[0]
    n_chunks = b_per_w // CHUNK
    wid = lax.axis_index("s") * NUM_CORES + lax.axis_index("c")
    base = wid * b_per_w

    bufs = [(h1_0, t1_0, h2_0, t2_0, r12_0, tt_0),
            (h1_1, t1_1, h2_1, t2_1, r12_1, tt_1)]
    sems = [sem0, sem1]

    pltpu.sync_copy(heads_hbm.at[pl.ds(base, b_per_w)], heads_v)
    pltpu.sync_copy(rels_hbm.at[pl.ds(base, b_per_w)], rels_v)
    pltpu.sync_copy(tails_hbm.at[pl.ds(base, b_per_w)], tails_v)
    pltpu.sync_copy(dates_hbm.at[pl.ds(base, b_per_w)], dates_v)

    def chunk_srcs(off, p):
        h_idx = heads_v.at[pl.ds(off, CHUNK)]
        t_idx = tails_v.at[pl.ds(off, CHUNK)]
        r_idx = rels_v.at[pl.ds(off, CHUNK)]
        d_idx = dates_v.at[pl.ds(off, CHUNK)]
        srcs = (ent_h_hbm.at[h_idx], ent_t_hbm.at[t_idx],
                ent_h_hbm.at[t_idx], ent_t_hbm.at[h_idx],
                rel_fi_hbm.at[r_idx],
                tim_hbm.at[d_idx])
        return [pltpu.make_async_copy(s, d, sems[p])
                for s, d in zip(srcs, bufs[p])]

    def start_chunk(g, p):
        for cp in chunk_srcs(g * CHUNK, p):
            cp.start()

    def drain_chunk(p):
        for cp in chunk_srcs(0, p):
            cp.wait()

    lane_iota = lax.iota(jnp.int32, LANES)

    def compute_group(gi, chunk_off, p):
        h1_v, t1_v, h2_v, t2_v, r12_v, tt_v = bufs[p]

        def elem_step(e, _):
            row = gi * LANES + e
            c = jnp.zeros((LANES,), jnp.float32)
            for k in range(EMB_DIM // LANES):
                sl = pl.ds(k * LANES, LANES)
                h1 = h1_v[row, sl]
                t1 = t1_v[row, sl]
                h2 = h2_v[row, sl]
                t2 = t2_v[row, sl]
                r1 = r12_v[row, sl]
                r2 = r12_v[row, pl.ds(EMB_DIM + k * LANES, LANES)]
                cc = (r1 * (h1 * t1 + h2 * t2)
                      + r2 * (h1 * t2 - h2 * t1))
                if k * LANES < T_DIM:
                    cc = cc * tt_v[row, sl]
                c = c + cc
            ts_v[pl.ds(e * TS_PITCH, LANES)] = c
            return 0

        lax.fori_loop(0, LANES, elem_step, 0, unroll=2)

        rows_ts = lane_iota * TS_PITCH

        def col_step(l, acc):
            return acc + plsc.load_gather(ts_v, [rows_ts + l])

        acc = lax.fori_loop(0, LANES, col_step,
                            jnp.zeros((LANES,), jnp.float32), unroll=4)
        out_v[pl.ds(chunk_off + gi * LANES, LANES)] = acc
        return chunk_off

    start_chunk(0, 0)

    def pair_body(i, _):
        for b in range(NBUF):
            g = i * NBUF + b
            p = b
            drain_chunk(p)

            @pl.when(g + 1 < n_chunks)
            def _():
                start_chunk(g + 1, 1 - p)

            lax.fori_loop(0, CHUNK // LANES,
                          functools.partial(compute_group, p=p),
                          g * CHUNK)
        return 0

    lax.fori_loop(0, n_chunks // NBUF, pair_body, 0)

    pltpu.sync_copy(out_v, out_hbm.at[pl.ds(base, b_per_w)])


def kernel(heads, rels, tails, dates, ent_embs_h, ent_embs_t,
           rel_embs_f, rel_embs_i, tim_embs_f):
    B = heads.shape[0]
    assert B % (8 * NW) == 0
    b_per_w = B // NW
    tim_full = jnp.concatenate(
        [tim_embs_f, jnp.ones_like(tim_embs_f)], axis=1)
    rel_fi = jnp.concatenate([rel_embs_f, rel_embs_i], axis=1)

    mesh = plsc.VectorSubcoreMesh(core_axis_name="c", subcore_axis_name="s")
    f = functools.partial(
        pl.kernel,
        mesh=mesh,
        compiler_params=pltpu.CompilerParams(needs_layout_passes=False),
        out_type=jax.ShapeDtypeStruct((B,), jnp.float32),
        scratch_types=[
            pltpu.VMEM((b_per_w,), jnp.int32),
            pltpu.VMEM((b_per_w,), jnp.int32),
            pltpu.VMEM((b_per_w,), jnp.int32),
            pltpu.VMEM((b_per_w,), jnp.int32),
            *[pltpu.VMEM((CHUNK, EMB_DIM), jnp.float32) for _ in range(4)]
            + [pltpu.VMEM((CHUNK, 2 * EMB_DIM), jnp.float32),
               pltpu.VMEM((CHUNK, EMB_DIM), jnp.float32)]
            + [pltpu.VMEM((CHUNK, EMB_DIM), jnp.float32) for _ in range(4)]
            + [pltpu.VMEM((CHUNK, 2 * EMB_DIM), jnp.float32),
               pltpu.VMEM((CHUNK, EMB_DIM), jnp.float32)],
            pltpu.VMEM((LANES * TS_PITCH,), jnp.float32),
            pltpu.VMEM((b_per_w,), jnp.float32),
            pltpu.SemaphoreType.DMA,
            pltpu.SemaphoreType.DMA,
        ],
    )(_tec_body)
    return f(heads, rels, tails, dates, ent_embs_h, ent_embs_t,
             rel_fi, tim_full)


# speedup vs baseline: 1.0591x; 1.0591x over previous
import functools

import jax
import jax.numpy as jnp
from jax import lax
from jax.experimental import pallas as pl
from jax.experimental.pallas import tpu as pltpu
from jax.experimental.pallas import tpu_sc as plsc

NUM_CORES = 2
NUM_SUBCORES = 16
NW = NUM_CORES * NUM_SUBCORES
LANES = 16

EMB_DIM = 128
T_DIM = 64
CHUNK = 64
NBUF = 2
NOPND = 7

TS_PITCH = 24


def _tec_body(heads_hbm, rels_hbm, tails_hbm, dates_hbm,
              ent_h_hbm, ent_t_hbm, rel_f_hbm, rel_i_hbm, tim_hbm,
              out_hbm,
              heads_v, rels_v, tails_v, dates_v,
              h1_0, t1_0, h2_0, t2_0, r1_0, r2_0, tt_0,
              h1_1, t1_1, h2_1, t2_1, r1_1, r2_1, tt_1,
              ts_v, out_v,
              sem0, sem1):
    b_per_w = heads_v.shape[0]
    n_chunks = b_per_w // CHUNK
    wid = lax.axis_index("s") * NUM_CORES + lax.axis_index("c")
    base = wid * b_per_w

    bufs = [(h1_0, t1_0, h2_0, t2_0, r1_0, r2_0, tt_0),
            (h1_1, t1_1, h2_1, t2_1, r1_1, r2_1, tt_1)]
    sems = [sem0, sem1]

    idx_cps = [
        pltpu.make_async_copy(heads_hbm.at[pl.ds(base, b_per_w)], heads_v,
                              sem0),
        pltpu.make_async_copy(rels_hbm.at[pl.ds(base, b_per_w)], rels_v,
                              sem0),
        pltpu.make_async_copy(tails_hbm.at[pl.ds(base, b_per_w)], tails_v,
                              sem0),
        pltpu.make_async_copy(dates_hbm.at[pl.ds(base, b_per_w)], dates_v,
                              sem0),
    ]
    for cp in idx_cps:
        cp.start()
    for cp in idx_cps:
        cp.wait()

    def chunk_srcs(off, p):
        h_idx = heads_v.at[pl.ds(off, CHUNK)]
        t_idx = tails_v.at[pl.ds(off, CHUNK)]
        r_idx = rels_v.at[pl.ds(off, CHUNK)]
        d_idx = dates_v.at[pl.ds(off, CHUNK)]
        srcs = (ent_h_hbm.at[h_idx], ent_t_hbm.at[t_idx],
                ent_h_hbm.at[t_idx], ent_t_hbm.at[h_idx],
                rel_f_hbm.at[r_idx], rel_i_hbm.at[r_idx],
                tim_hbm.at[d_idx])
        return [pltpu.make_async_copy(s, d, sems[p])
                for s, d in zip(srcs, bufs[p])]

    def start_chunk(g, p):
        for cp in chunk_srcs(g * CHUNK, p):
            cp.start()

    def drain_chunk(p):
        for cp in chunk_srcs(0, p):
            cp.wait()

    lane_iota = lax.iota(jnp.int32, LANES)

    def compute_group(gi, chunk_off, p):
        h1_v, t1_v, h2_v, t2_v, r1_v, r2_v, tt_v = bufs[p]

        def elem_step(e, _):
            row = gi * LANES + e
            c = jnp.zeros((LANES,), jnp.float32)
            for k in range(EMB_DIM // LANES):
                sl = pl.ds(k * LANES, LANES)
                h1 = h1_v[row, sl]
                t1 = t1_v[row, sl]
                h2 = h2_v[row, sl]
                t2 = t2_v[row, sl]
                r1 = r1_v[row, sl]
                r2 = r2_v[row, sl]
                cc = (r1 * (h1 * t1 + h2 * t2)
                      + r2 * (h1 * t2 - h2 * t1))
                if k * LANES < T_DIM:
                    cc = cc * tt_v[row, sl]
                c = c + cc
            ts_v[pl.ds(e * TS_PITCH, LANES)] = c
            return 0

        lax.fori_loop(0, LANES, elem_step, 0, unroll=2)

        acc = jnp.zeros((LANES,), jnp.float32)
        rows_ts = lane_iota * TS_PITCH
        for l in range(LANES):
            acc = acc + plsc.load_gather(ts_v, [rows_ts + l])
        out_v[pl.ds(chunk_off + gi * LANES, LANES)] = acc
        return chunk_off

    start_chunk(0, 0)

    def pair_body(i, _):
        for b in range(NBUF):
            g = i * NBUF + b
            p = b
            drain_chunk(p)

            @pl.when(g + 1 < n_chunks)
            def _():
                start_chunk(g + 1, 1 - p)

            lax.fori_loop(0, CHUNK // LANES,
                          functools.partial(compute_group, p=p),
                          g * CHUNK)
        return 0

    lax.fori_loop(0, n_chunks // NBUF, pair_body, 0)

    pltpu.sync_copy(out_v, out_hbm.at[pl.ds(base, b_per_w)])


def kernel(heads, rels, tails, dates, ent_embs_h, ent_embs_t,
           rel_embs_f, rel_embs_i, tim_embs_f):
    B = heads.shape[0]
    assert B % (8 * NW) == 0
    b_per_w = B // NW
    tim_full = jnp.concatenate(
        [tim_embs_f, jnp.ones_like(tim_embs_f)], axis=1)

    mesh = plsc.VectorSubcoreMesh(core_axis_name="c", subcore_axis_name="s")
    f = functools.partial(
        pl.kernel,
        mesh=mesh,
        compiler_params=pltpu.CompilerParams(needs_layout_passes=False),
        out_type=jax.ShapeDtypeStruct((B,), jnp.float32),
        scratch_types=[
            pltpu.VMEM((b_per_w,), jnp.int32),
            pltpu.VMEM((b_per_w,), jnp.int32),
            pltpu.VMEM((b_per_w,), jnp.int32),
            pltpu.VMEM((b_per_w,), jnp.int32),
            *[pltpu.VMEM((CHUNK, EMB_DIM), jnp.float32)
              for _ in range(NOPND * NBUF)],
            pltpu.VMEM((LANES * TS_PITCH,), jnp.float32),
            pltpu.VMEM((b_per_w,), jnp.float32),
            pltpu.SemaphoreType.DMA,
            pltpu.SemaphoreType.DMA,
        ],
    )(_tec_body)
    return f(heads, rels, tails, dates, ent_embs_h, ent_embs_t,
             rel_embs_f, rel_embs_i, tim_full)
